# baseline (speedup 1.0000x reference)
import dataclasses
import functools

import jax
import jax.numpy as jnp
from jax import lax
from jax.experimental import pallas as pl
from jax.experimental.pallas import tpu as pltpu
from jax.experimental.pallas import tpu_sc as plsc

N = 10000
E = 320000
D = 128
DP = D // 2
BLK = 64
NC = 2
NS = 16
NW = NC * NS
WBLK = 160
NBLK = WBLK * NW
STRIPE = 624
TAIL = N - STRIPE * NS
ZR = 24

_mesh = plsc.VectorSubcoreMesh(core_axis_name="c", subcore_axis_name="s")

_cp = pltpu.CompilerParams()
if "needs_layout_passes" in pltpu.CompilerParams.__dataclass_fields__:
    _cp = dataclasses.replace(_cp, needs_layout_passes=False)


@functools.partial(
    pl.kernel,
    mesh=_mesh,
    out_type=jax.ShapeDtypeStruct((NC, N, D), jnp.float32),
    scratch_types=[
        [pltpu.VMEM((1, BLK), jnp.int32) for _ in range(4)],
        [pltpu.VMEM((1, BLK), jnp.int32) for _ in range(4)],
        [pltpu.VMEM((1, BLK), jnp.float32) for _ in range(4)],
        [pltpu.VMEM((BLK, D), jnp.float32) for _ in range(4)],
        pltpu.VMEM((ZR, D), jnp.float32),
        pltpu.VMEM_SHARED((N, D), jnp.float32),
        [pltpu.SemaphoreType.DMA for _ in range(4)],
        [pltpu.SemaphoreType.DMA for _ in range(4)],
        [pltpu.SemaphoreType.DMA for _ in range(4)],
        [pltpu.SemaphoreType.DMA for _ in range(4)],
    ],
    compiler_params=_cp,
)
def _sc_pass(h_hbm, col_hbm, row_hbm, val_hbm, out_hbm,
             colb, rowb, valb, gv, zb, acc, gs, cfs, fs, as_):
    c = lax.axis_index("c")
    s = lax.axis_index("s")
    w = c * NS + s
    bstart = WBLK * w

    def gather_start(j, m):
        pltpu.async_copy(h_hbm.at[colb[m % 4].at[0]], gv[m % 4], gs[m % 4])

    def gather_wait(j, m):
        pltpu.make_async_copy(h_hbm.at[colb[m % 4].at[0]], gv[m % 4],
                              gs[m % 4]).wait()

    def cfetch_start(j, m):
        pltpu.async_copy(col_hbm.at[pl.ds(bstart + j, 1)], colb[m % 4],
                         cfs[m % 4])

    def cfetch_wait(j, m):
        pltpu.make_async_copy(col_hbm.at[pl.ds(bstart + j, 1)], colb[m % 4],
                              cfs[m % 4]).wait()

    def fetch_start(j, m):
        pltpu.async_copy(row_hbm.at[pl.ds(bstart + j, 1)], rowb[m % 4],
                         fs[m % 4])
        pltpu.async_copy(val_hbm.at[pl.ds(bstart + j, 1)], valb[m % 4],
                         fs[m % 4])

    def fetch_wait(j, m):
        pltpu.make_async_copy(row_hbm.at[pl.ds(bstart + j, 1)], rowb[m % 4],
                              fs[m % 4]).wait()
        pltpu.make_async_copy(val_hbm.at[pl.ds(bstart + j, 1)], valb[m % 4],
                              fs[m % 4]).wait()

    def scatter_start(m):
        pltpu.async_copy(gv[m % 4], acc.at[rowb[m % 4].at[0]], as_[m % 4],
                         add=True)

    def scatter_wait(m):
        pltpu.make_async_copy(gv[m % 4], acc.at[rowb[m % 4].at[0]],
                              as_[m % 4]).wait()

    def scale(m):
        g, v = gv[m % 4], valb[m % 4]

        @pl.loop(0, BLK // 16, unroll=4)
        def _(grp):
            vvs = v[0, pl.ds(grp * 16, 16)]
            for jj in range(16):
                vv = vvs.at[lax.broadcast(jj, (16,))].get(
                    mode="promise_in_bounds")
                e = grp * 16 + jj
                for d in range(D // 16):
                    sl = (e, pl.ds(d * 16, 16))
                    g[sl] = g[sl] * vv

    nb = jnp.clip(E // BLK - w * WBLK, 0, WBLK)

    for m in range(4):
        cfetch_start(m, m)
        fetch_start(m, m)
    for m in range(2):
        cfetch_wait(m, m)
        gather_start(m, m)

    @pl.loop(0, ZR)
    def _(r):
        for d in range(D // 16):
            zb[r, pl.ds(d * 16, 16)] = jnp.zeros((16,), jnp.float32)

    for t in range(STRIPE // ZR):
        pltpu.sync_copy(zb, acc.at[pl.ds(s * STRIPE + t * ZR, ZR)])

    @pl.when(s == NS - 1)
    def _():
        pltpu.sync_copy(zb.at[pl.ds(0, TAIL)], acc.at[pl.ds(STRIPE * NS, TAIL)])

    plsc.subcore_barrier()

    @pl.loop(0, nb, step=4)
    def _(j):
        for m in range(4):
            gather_wait(j + m, m)
            fetch_wait(j + m, m)
            if m < 2:
                @pl.when(j >= 2)
                def _():
                    scatter_wait(m + 2)
            else:
                scatter_wait(m + 2)

            @pl.when(j + m + 4 < nb)
            def _():
                cfetch_start(j + m + 4, m)

            @pl.when(j + m + 2 < nb)
            def _():
                cfetch_wait(j + m + 2, m + 2)
                gather_start(j + m + 2, m + 2)
                fetch_start(j + m + 2, m + 2)

            scale(m)
            scatter_start(m)

    scatter_wait(2)
    scatter_wait(3)

    plsc.subcore_barrier()
    pltpu.sync_copy(acc.at[pl.ds(s * STRIPE, STRIPE)],
                    out_hbm.at[c].at[pl.ds(s * STRIPE, STRIPE)])

    @pl.when(s == NS - 1)
    def _():
        pltpu.sync_copy(acc.at[pl.ds(STRIPE * NS, TAIL)],
                        out_hbm.at[c].at[pl.ds(STRIPE * NS, TAIL)])


def _merge_body(p_ref, o_ref):
    o_ref[...] = p_ref[0] + p_ref[1]


def _merge(parts):
    return pl.pallas_call(
        _merge_body,
        out_shape=jax.ShapeDtypeStruct((N, D), jnp.float32),
    )(parts)


def kernel(x, edge_row, edge_col, edge_val):
    pad = NBLK * BLK - E
    row = jnp.pad(edge_row.astype(jnp.int32), (0, pad)).reshape(NBLK, BLK)
    col = jnp.pad(edge_col.astype(jnp.int32), (0, pad)).reshape(NBLK, BLK)
    val = jnp.pad(edge_val, (0, pad)).reshape(NBLK, BLK)
    parts1 = _sc_pass(x, col, row, val)
    parts2 = _sc_pass(_merge(parts1), col, row, val)
    return _merge(parts2)


# speedup vs baseline: 1.2206x; 1.2206x over previous
import dataclasses
import functools

import jax
import jax.numpy as jnp
from jax import lax
from jax.experimental import pallas as pl
from jax.experimental.pallas import tpu as pltpu
from jax.experimental.pallas import tpu_sc as plsc

N = 10000
E = 320000
D = 128
DP = D // 2
BLK = 64
NC = 2
NS = 16
NW = NC * NS
WBLK = 160
NBLK = WBLK * NW
STRIPE = 624
TAIL = N - STRIPE * NS
ZR = 24

_mesh = plsc.VectorSubcoreMesh(core_axis_name="c", subcore_axis_name="s")

_cp = pltpu.CompilerParams()
if "needs_layout_passes" in pltpu.CompilerParams.__dataclass_fields__:
    _cp = dataclasses.replace(_cp, needs_layout_passes=False)


@functools.partial(
    pl.kernel,
    mesh=_mesh,
    out_type=jax.ShapeDtypeStruct((NC, N, D), jnp.float32),
    scratch_types=[
        [pltpu.VMEM((1, BLK), jnp.int32) for _ in range(4)],
        [pltpu.VMEM((1, BLK), jnp.int32) for _ in range(4)],
        [pltpu.VMEM((1, BLK), jnp.float32) for _ in range(4)],
        [pltpu.VMEM((BLK, D), jnp.float32) for _ in range(4)],
        pltpu.VMEM((ZR, D), jnp.float32),
        pltpu.VMEM_SHARED((N, D), jnp.float32),
        [pltpu.SemaphoreType.DMA for _ in range(4)],
        [pltpu.SemaphoreType.DMA for _ in range(4)],
        [pltpu.SemaphoreType.DMA for _ in range(4)],
        [pltpu.SemaphoreType.DMA for _ in range(4)],
    ],
    compiler_params=_cp,
)
def _sc_pass(h_hbm, col_hbm, row_hbm, val_hbm, out_hbm,
             colb, rowb, valb, gv, zb, acc, gs, cfs, fs, as_):
    c = lax.axis_index("c")
    s = lax.axis_index("s")
    w = c * NS + s
    bstart = WBLK * w

    def gather_start(j, m):
        pltpu.async_copy(h_hbm.at[colb[m % 4].at[0]], gv[m % 4], gs[m % 4])

    def gather_wait(j, m):
        pltpu.make_async_copy(h_hbm.at[colb[m % 4].at[0]], gv[m % 4],
                              gs[m % 4]).wait()

    def cfetch_start(j, m):
        pltpu.async_copy(col_hbm.at[pl.ds(bstart + j, 1)], colb[m % 4],
                         cfs[m % 4])

    def cfetch_wait(j, m):
        pltpu.make_async_copy(col_hbm.at[pl.ds(bstart + j, 1)], colb[m % 4],
                              cfs[m % 4]).wait()

    def fetch_start(j, m):
        pltpu.async_copy(row_hbm.at[pl.ds(bstart + j, 1)], rowb[m % 4],
                         fs[m % 4])
        pltpu.async_copy(val_hbm.at[pl.ds(bstart + j, 1)], valb[m % 4],
                         fs[m % 4])

    def fetch_wait(j, m):
        pltpu.make_async_copy(row_hbm.at[pl.ds(bstart + j, 1)], rowb[m % 4],
                              fs[m % 4]).wait()
        pltpu.make_async_copy(val_hbm.at[pl.ds(bstart + j, 1)], valb[m % 4],
                              fs[m % 4]).wait()

    def scatter_start(m):
        pltpu.async_copy(gv[m % 4], acc.at[rowb[m % 4].at[0]], as_[m % 4],
                         add=True)

    def scatter_wait(m):
        pltpu.make_async_copy(gv[m % 4], acc.at[rowb[m % 4].at[0]],
                              as_[m % 4]).wait()

    def scale(m):
        g, v = gv[m % 4], valb[m % 4]

        @pl.loop(0, BLK // 16)
        def _(grp):
            vvs = v[0, pl.ds(grp * 16, 16)]
            for jj in range(16):
                vv = vvs.at[lax.broadcast(jj, (16,))].get(
                    mode="promise_in_bounds")
                e = grp * 16 + jj
                for d in range(D // 16):
                    sl = (e, pl.ds(d * 16, 16))
                    g[sl] = g[sl] * vv

    nb = jnp.clip(E // BLK - w * WBLK, 0, WBLK)

    for m in range(4):
        cfetch_start(m, m)
        fetch_start(m, m)
    for m in range(2):
        cfetch_wait(m, m)
        gather_start(m, m)

    @pl.loop(0, ZR)
    def _(r):
        for d in range(D // 16):
            zb[r, pl.ds(d * 16, 16)] = jnp.zeros((16,), jnp.float32)

    for t in range(STRIPE // ZR):
        pltpu.sync_copy(zb, acc.at[pl.ds(s * STRIPE + t * ZR, ZR)])

    @pl.when(s == NS - 1)
    def _():
        pltpu.sync_copy(zb.at[pl.ds(0, TAIL)], acc.at[pl.ds(STRIPE * NS, TAIL)])

    plsc.subcore_barrier()

    @pl.loop(0, nb, step=4)
    def _(j):
        for m in range(4):
            gather_wait(j + m, m)
            fetch_wait(j + m, m)
            if m < 2:
                @pl.when(j >= 2)
                def _():
                    scatter_wait(m + 2)
            else:
                scatter_wait(m + 2)

            @pl.when(j + m + 4 < nb)
            def _():
                cfetch_start(j + m + 4, m)

            @pl.when(j + m + 2 < nb)
            def _():
                cfetch_wait(j + m + 2, m + 2)
                gather_start(j + m + 2, m + 2)
                fetch_start(j + m + 2, m + 2)

            scale(m)
            scatter_start(m)

    scatter_wait(2)
    scatter_wait(3)

    plsc.subcore_barrier()
    pltpu.sync_copy(acc.at[pl.ds(s * STRIPE, STRIPE)],
                    out_hbm.at[c].at[pl.ds(s * STRIPE, STRIPE)])

    @pl.when(s == NS - 1)
    def _():
        pltpu.sync_copy(acc.at[pl.ds(STRIPE * NS, TAIL)],
                        out_hbm.at[c].at[pl.ds(STRIPE * NS, TAIL)])


def _merge_body(p_ref, o_ref):
    o_ref[...] = p_ref[0] + p_ref[1]


def _merge(parts):
    return pl.pallas_call(
        _merge_body,
        out_shape=jax.ShapeDtypeStruct((N, D), jnp.float32),
    )(parts)


def kernel(x, edge_row, edge_col, edge_val):
    pad = NBLK * BLK - E
    row = jnp.pad(edge_row.astype(jnp.int32), (0, pad)).reshape(NBLK, BLK)
    col = jnp.pad(edge_col.astype(jnp.int32), (0, pad)).reshape(NBLK, BLK)
    val = jnp.pad(edge_val, (0, pad)).reshape(NBLK, BLK)
    parts1 = _sc_pass(x, col, row, val)
    parts2 = _sc_pass(_merge(parts1), col, row, val)
    return _merge(parts2)
